# baseline (speedup 1.0000x reference)
import functools

import jax
import jax.numpy as jnp
from jax import lax
from jax.experimental import pallas as pl
from jax.experimental.pallas import tpu as pltpu
from jax.experimental.pallas import tpu_sc as plsc

_B = 16384
_D = 16
_H = 16
_NC = 2
_NS = 16
_NW = _NC * _NS
_BPW = _B // _NW
_CHUNK = 128
_NCH = _BPW // _CHUNK
_GPC = _CHUNK // 16
_NSLAB = 1000000 * _D // 128


def _wrow(m):
    return m // 8, (m % 8) * 16


def _sc_body(uidx_hbm, eidx_hbm, utab_hbm, etab_hbm, wpack_hbm, out_hbm,
             uidx_v, eidx_v, uslab_v, eslab_v,
             ubufA, ubufB, ebufA, ebufB, wpack_v, out_v,
             semA_u, semB_u, semA_e, semB_e):
    wid = lax.axis_index("s") * _NC + lax.axis_index("c")
    base = wid * _BPW
    pltpu.sync_copy(uidx_hbm.at[pl.ds(base, _BPW)], uidx_v)
    pltpu.sync_copy(eidx_hbm.at[pl.ds(base, _BPW)], eidx_v)
    pltpu.sync_copy(wpack_hbm, wpack_v)
    for t in range(_BPW // 16):
        sl = pl.ds(t * 16, 16)
        uslab_v[sl] = lax.shift_right_logical(uidx_v[sl], 3)
        eslab_v[sl] = lax.shift_right_logical(eidx_v[sl], 3)

    ubufs = (ubufA, ubufB)
    ebufs = (ebufA, ebufB)
    usems = (semA_u, semB_u)
    esems = (semA_e, semB_e)

    def fire(c):
        sl = pl.ds(c * _CHUNK, _CHUNK)
        cu = pltpu.async_copy(
            utab_hbm.at[uslab_v.at[sl]], ubufs[c % 2], usems[c % 2])
        ce = pltpu.async_copy(
            etab_hbm.at[eslab_v.at[sl]], ebufs[c % 2], esems[c % 2])
        return cu, ce

    inflight = fire(0)

    def make_group(c):
        ubuf, ebuf = ubufs[c % 2], ebufs[c % 2]

        def group(g, _):
            s0 = c * _CHUNK + g * 16
            lrow = g * 16 + lax.iota(jnp.int32, 16)
            idxu = uidx_v[pl.ds(s0, 16)]
            idxe = eidx_v[pl.ds(s0, 16)]
            cu = (idxu & 7) * 16
            ce = (idxe & 7) * 16
            xs = []
            for k in range(_D):
                xs.append(plsc.load_gather(ubuf, [lrow, cu + k]))
            for k in range(_D):
                xs.append(plsc.load_gather(ebuf, [lrow, ce + k]))
            r, c0 = _wrow(544)
            o = wpack_v[r, pl.ds(c0, 16)]
            for j in range(_H):
                r, c0 = _wrow(512 + j)
                acc = wpack_v[r, pl.ds(c0, 16)]
                for k in range(2 * _D):
                    r, c0 = _wrow(j * 2 * _D + k)
                    acc = acc + wpack_v[r, pl.ds(c0, 16)] * xs[k]
                r, c0 = _wrow(528 + j)
                o = o + wpack_v[r, pl.ds(c0, 16)] * jnp.maximum(acc, 0.0)
            out_v[pl.ds(s0, 16)] = 1.0 / (1.0 + jnp.exp(-o))
            return 0

        return group

    for c in range(_NCH):
        cu, ce = inflight
        if c + 1 < _NCH:
            nxt = fire(c + 1)
        cu.wait()
        ce.wait()
        if c + 1 < _NCH:
            inflight = nxt
        lax.fori_loop(0, _GPC, make_group(c), 0)

    pltpu.sync_copy(out_v, out_hbm.at[pl.ds(base, _BPW)])


_sc_ncf = functools.partial(
    pl.kernel,
    out_type=jax.ShapeDtypeStruct((_B,), jnp.float32),
    mesh=plsc.VectorSubcoreMesh(core_axis_name="c", subcore_axis_name="s"),
    compiler_params=pltpu.CompilerParams(needs_layout_passes=False),
    scratch_types=[
        pltpu.VMEM((_BPW,), jnp.int32),
        pltpu.VMEM((_BPW,), jnp.int32),
        pltpu.VMEM((_BPW,), jnp.int32),
        pltpu.VMEM((_BPW,), jnp.int32),
        pltpu.VMEM((_CHUNK, 128), jnp.float32),
        pltpu.VMEM((_CHUNK, 128), jnp.float32),
        pltpu.VMEM((_CHUNK, 128), jnp.float32),
        pltpu.VMEM((_CHUNK, 128), jnp.float32),
        pltpu.VMEM((72, 128), jnp.float32),
        pltpu.VMEM((_BPW,), jnp.float32),
        pltpu.SemaphoreType.DMA,
        pltpu.SemaphoreType.DMA,
        pltpu.SemaphoreType.DMA,
        pltpu.SemaphoreType.DMA,
    ],
)(_sc_body)


def kernel(user, event, user_emb, event_emb, W1, b1, W2, b2):
    utab = user_emb.reshape(_NSLAB, 128)
    etab = event_emb.reshape(_NSLAB, 128)
    params = jnp.concatenate(
        [W1.reshape(-1), b1.reshape(-1), W2.reshape(-1), b2.reshape(-1)])
    splats = jnp.broadcast_to(params.reshape(-1, 1), (params.shape[0], 16))
    wpack = jnp.pad(splats.reshape(-1), (0, 72 * 128 - params.shape[0] * 16))
    wpack = wpack.reshape(72, 128)
    out = _sc_ncf(user, event, utab, etab, wpack)
    return out.reshape(_B, 1)


# speedup vs baseline: 1.3514x; 1.3514x over previous
import functools

import jax
import jax.numpy as jnp
from jax import lax
from jax.experimental import pallas as pl
from jax.experimental.pallas import tpu as pltpu
from jax.experimental.pallas import tpu_sc as plsc

_B = 16384
_D = 16
_H = 16
_NC = 2
_NS = 16
_NW = _NC * _NS
_BPW = _B // _NW
_CHUNK = 128
_NCH = _BPW // _CHUNK
_GPC = _CHUNK // 16
_NSLAB = 1000000 * _D // 128


def _wrow(m):
    return m // 8, (m % 8) * 16


def _sc_body(uidx_hbm, eidx_hbm, utab_hbm, etab_hbm, wpack_hbm, out_hbm,
             uidx_v, eidx_v, uslab_v, eslab_v,
             ubufA, ubufB, ebufA, ebufB, wpack_v, out_v,
             semA_u, semB_u, semA_e, semB_e):
    wid = lax.axis_index("s") * _NC + lax.axis_index("c")
    base = wid * _BPW
    pltpu.sync_copy(uidx_hbm.at[pl.ds(base, _BPW)], uidx_v)
    pltpu.sync_copy(eidx_hbm.at[pl.ds(base, _BPW)], eidx_v)
    pltpu.sync_copy(wpack_hbm, wpack_v)
    for t in range(_BPW // 16):
        sl = pl.ds(t * 16, 16)
        uslab_v[sl] = lax.shift_right_logical(uidx_v[sl], 3)
        eslab_v[sl] = lax.shift_right_logical(eidx_v[sl], 3)

    ubufs = (ubufA, ubufB)
    ebufs = (ebufA, ebufB)
    usems = (semA_u, semB_u)
    esems = (semA_e, semB_e)

    def fire(c):
        sl = pl.ds(c * _CHUNK, _CHUNK)
        cu = pltpu.async_copy(
            utab_hbm.at[uslab_v.at[sl]], ubufs[c % 2], usems[c % 2])
        ce = pltpu.async_copy(
            etab_hbm.at[eslab_v.at[sl]], ebufs[c % 2], esems[c % 2])
        return cu, ce

    inflight = fire(0)

    def make_group(c):
        ubuf, ebuf = ubufs[c % 2], ebufs[c % 2]

        def group(g, _):
            s0 = c * _CHUNK + g * 16
            lrow = g * 16 + lax.iota(jnp.int32, 16)
            idxu = uidx_v[pl.ds(s0, 16)]
            idxe = eidx_v[pl.ds(s0, 16)]
            cu = (idxu & 7) * 16
            ce = (idxe & 7) * 16
            xs = []
            for k in range(_D):
                xs.append(plsc.load_gather(ubuf, [lrow, cu + k]))
            for k in range(_D):
                xs.append(plsc.load_gather(ebuf, [lrow, ce + k]))
            r, c0 = _wrow(544)
            o = wpack_v[r, pl.ds(c0, 16)]
            for j in range(_H):
                r, c0 = _wrow(512 + j)
                acc = wpack_v[r, pl.ds(c0, 16)]
                for k in range(2 * _D):
                    r, c0 = _wrow(j * 2 * _D + k)
                    acc = acc + wpack_v[r, pl.ds(c0, 16)] * xs[k]
                r, c0 = _wrow(528 + j)
                o = o + wpack_v[r, pl.ds(c0, 16)] * jnp.maximum(acc, 0.0)
            out_v[pl.ds(s0, 16)] = 1.0 / (1.0 + jnp.exp(-o))
            return 0

        return group

    for c in range(_NCH):
        cu, ce = inflight
        if c + 1 < _NCH:
            nxt = fire(c + 1)
        cu.wait()
        ce.wait()
        if c + 1 < _NCH:
            inflight = nxt
        lax.fori_loop(0, _GPC, make_group(c), 0)

    pltpu.sync_copy(out_v, out_hbm.at[pl.ds(base, _BPW)])


_sc_ncf = functools.partial(
    pl.kernel,
    out_type=jax.ShapeDtypeStruct((_B,), jnp.float32),
    mesh=plsc.VectorSubcoreMesh(core_axis_name="c", subcore_axis_name="s"),
    compiler_params=pltpu.CompilerParams(needs_layout_passes=False),
    scratch_types=[
        pltpu.VMEM((_BPW,), jnp.int32),
        pltpu.VMEM((_BPW,), jnp.int32),
        pltpu.VMEM((_BPW,), jnp.int32),
        pltpu.VMEM((_BPW,), jnp.int32),
        pltpu.VMEM((_CHUNK, 128), jnp.float32),
        pltpu.VMEM((_CHUNK, 128), jnp.float32),
        pltpu.VMEM((_CHUNK, 128), jnp.float32),
        pltpu.VMEM((_CHUNK, 128), jnp.float32),
        pltpu.VMEM((72, 128), jnp.float32),
        pltpu.VMEM((_BPW,), jnp.float32),
        pltpu.SemaphoreType.DMA,
        pltpu.SemaphoreType.DMA,
        pltpu.SemaphoreType.DMA,
        pltpu.SemaphoreType.DMA,
    ],
)(_sc_body)


_RC = 8192
_RB = (1000000 + _RC - 1) // _RC


def _repack_body(ut_ref, et_ref, sel_ref, uo_ref, eo_ref):
    def one(t_ref, o_ref):
        y = t_ref[...].T.reshape(_RC // 8, 8, _D)
        acc = jnp.dot(y[:, 0, :], sel_ref[0],
                      preferred_element_type=jnp.float32)
        for s in range(1, 8):
            acc = acc + jnp.dot(y[:, s, :], sel_ref[s],
                                preferred_element_type=jnp.float32)
        o_ref[...] = acc

    one(ut_ref, uo_ref)
    one(et_ref, eo_ref)


_tc_repack = pl.pallas_call(
    _repack_body,
    grid=(_RB,),
    in_specs=[
        pl.BlockSpec((_D, _RC), lambda i: (0, i)),
        pl.BlockSpec((_D, _RC), lambda i: (0, i)),
        pl.BlockSpec((8, _D, 128), lambda i: (0, 0, 0)),
    ],
    out_specs=[
        pl.BlockSpec((_RC // 8, 128), lambda i: (i, 0)),
        pl.BlockSpec((_RC // 8, 128), lambda i: (i, 0)),
    ],
    out_shape=[
        jax.ShapeDtypeStruct((_NSLAB, 128), jnp.float32),
        jax.ShapeDtypeStruct((_NSLAB, 128), jnp.float32),
    ],
)

_SEL = None


def _sel_const():
    global _SEL
    if _SEL is None:
        import numpy as _np
        m = _np.zeros((8, _D, 128), _np.float32)
        for s in range(8):
            for d in range(_D):
                m[s, d, 16 * s + d] = 1.0
        _SEL = jnp.asarray(m)
    return _SEL


def kernel(user, event, user_emb, event_emb, W1, b1, W2, b2):
    utab, etab = _tc_repack(user_emb.T, event_emb.T, _sel_const())
    params = jnp.concatenate(
        [W1.reshape(-1), b1.reshape(-1), W2.reshape(-1), b2.reshape(-1)])
    splats = jnp.broadcast_to(params.reshape(-1, 1), (params.shape[0], 16))
    wpack = jnp.pad(splats.reshape(-1), (0, 72 * 128 - params.shape[0] * 16))
    wpack = wpack.reshape(72, 128)
    out = _sc_ncf(user, event, utab, etab, wpack)
    return out.reshape(_B, 1)


# speedup vs baseline: 2.4360x; 1.8026x over previous
import functools

import jax
import jax.numpy as jnp
from jax import lax
from jax.experimental import pallas as pl
from jax.experimental.pallas import tpu as pltpu
from jax.experimental.pallas import tpu_sc as plsc

_B = 16384
_D = 16
_H = 16
_NC = 2
_NS = 16
_NW = _NC * _NS
_BPW = _B // _NW
_CHUNK = 128
_NCH = _BPW // _CHUNK
_GPC = _CHUNK // 16
_NSLAB = 123 * 1024


def _wrow(m):
    return m // 8, (m % 8) * 16


def _sc_body(uidx_hbm, eidx_hbm, utab_hbm, etab_hbm, wpack_hbm, out_hbm,
             uidx_v, eidx_v, uslab_v, eslab_v,
             ubufA, ubufB, ebufA, ebufB, wpack_v, out_v,
             semA_u, semB_u, semA_e, semB_e):
    wid = lax.axis_index("s") * _NC + lax.axis_index("c")
    base = wid * _BPW
    pltpu.sync_copy(uidx_hbm.at[pl.ds(base, _BPW)], uidx_v)
    pltpu.sync_copy(eidx_hbm.at[pl.ds(base, _BPW)], eidx_v)
    pltpu.sync_copy(wpack_hbm, wpack_v)
    for t in range(_BPW // 16):
        sl = pl.ds(t * 16, 16)
        u = uidx_v[sl]
        e = eidx_v[sl]
        uslab_v[sl] = lax.shift_left(
            lax.shift_right_logical(u, 13), 10) + (u & 1023)
        eslab_v[sl] = lax.shift_left(
            lax.shift_right_logical(e, 13), 10) + (e & 1023)

    ubufs = (ubufA, ubufB)
    ebufs = (ebufA, ebufB)
    usems = (semA_u, semB_u)
    esems = (semA_e, semB_e)

    def fire(c):
        sl = pl.ds(c * _CHUNK, _CHUNK)
        cu = pltpu.async_copy(
            utab_hbm.at[uslab_v.at[sl]], ubufs[c % 2], usems[c % 2])
        ce = pltpu.async_copy(
            etab_hbm.at[eslab_v.at[sl]], ebufs[c % 2], esems[c % 2])
        return cu, ce

    inflight = fire(0)

    def make_group(c):
        ubuf, ebuf = ubufs[c % 2], ebufs[c % 2]

        def group(g, _):
            s0 = c * _CHUNK + g * 16
            lrow = g * 16 + lax.iota(jnp.int32, 16)
            idxu = uidx_v[pl.ds(s0, 16)]
            idxe = eidx_v[pl.ds(s0, 16)]
            cu = (lax.shift_right_logical(idxu, 10) & 7) * 16
            ce = (lax.shift_right_logical(idxe, 10) & 7) * 16
            xs = []
            for k in range(_D):
                xs.append(plsc.load_gather(ubuf, [lrow, cu + k]))
            for k in range(_D):
                xs.append(plsc.load_gather(ebuf, [lrow, ce + k]))
            r, c0 = _wrow(544)
            o = wpack_v[r, pl.ds(c0, 16)]
            for j in range(_H):
                r, c0 = _wrow(512 + j)
                acc = wpack_v[r, pl.ds(c0, 16)]
                for k in range(2 * _D):
                    r, c0 = _wrow(j * 2 * _D + k)
                    acc = acc + wpack_v[r, pl.ds(c0, 16)] * xs[k]
                r, c0 = _wrow(528 + j)
                o = o + wpack_v[r, pl.ds(c0, 16)] * jnp.maximum(acc, 0.0)
            out_v[pl.ds(s0, 16)] = 1.0 / (1.0 + jnp.exp(-o))
            return 0

        return group

    for c in range(_NCH):
        cu, ce = inflight
        if c + 1 < _NCH:
            nxt = fire(c + 1)
        cu.wait()
        ce.wait()
        if c + 1 < _NCH:
            inflight = nxt
        lax.fori_loop(0, _GPC, make_group(c), 0)

    pltpu.sync_copy(out_v, out_hbm.at[pl.ds(base, _BPW)])


_sc_ncf = functools.partial(
    pl.kernel,
    out_type=jax.ShapeDtypeStruct((_B,), jnp.float32),
    mesh=plsc.VectorSubcoreMesh(core_axis_name="c", subcore_axis_name="s"),
    compiler_params=pltpu.CompilerParams(needs_layout_passes=False),
    scratch_types=[
        pltpu.VMEM((_BPW,), jnp.int32),
        pltpu.VMEM((_BPW,), jnp.int32),
        pltpu.VMEM((_BPW,), jnp.int32),
        pltpu.VMEM((_BPW,), jnp.int32),
        pltpu.VMEM((_CHUNK, 128), jnp.float32),
        pltpu.VMEM((_CHUNK, 128), jnp.float32),
        pltpu.VMEM((_CHUNK, 128), jnp.float32),
        pltpu.VMEM((_CHUNK, 128), jnp.float32),
        pltpu.VMEM((72, 128), jnp.float32),
        pltpu.VMEM((_BPW,), jnp.float32),
        pltpu.SemaphoreType.DMA,
        pltpu.SemaphoreType.DMA,
        pltpu.SemaphoreType.DMA,
        pltpu.SemaphoreType.DMA,
    ],
)(_sc_body)


_RC = 8192
_RB = (1000000 + _RC - 1) // _RC


def _repack_body(ut_ref, et_ref, sel_ref, uo_ref, eo_ref):
    dn = (((0,), (0,)), ((), ()))

    def one(t_ref, o_ref):
        x = t_ref[...]
        q = _RC // 8
        acc = lax.dot_general(x[:, 0:q], sel_ref[0], dn,
                              preferred_element_type=jnp.float32)
        for s in range(1, 8):
            acc = acc + lax.dot_general(x[:, s * q:(s + 1) * q], sel_ref[s],
                                        dn,
                                        preferred_element_type=jnp.float32)
        o_ref[...] = acc

    one(ut_ref, uo_ref)
    one(et_ref, eo_ref)


_tc_repack = pl.pallas_call(
    _repack_body,
    grid=(_RB,),
    compiler_params=pltpu.CompilerParams(fuse_transposed_lhs_in_matmul=True),
    in_specs=[
        pl.BlockSpec((_D, _RC), lambda i: (0, i)),
        pl.BlockSpec((_D, _RC), lambda i: (0, i)),
        pl.BlockSpec((8, _D, 128), lambda i: (0, 0, 0)),
    ],
    out_specs=[
        pl.BlockSpec((_RC // 8, 128), lambda i: (i, 0)),
        pl.BlockSpec((_RC // 8, 128), lambda i: (i, 0)),
    ],
    out_shape=[
        jax.ShapeDtypeStruct((_NSLAB, 128), jnp.float32),
        jax.ShapeDtypeStruct((_NSLAB, 128), jnp.float32),
    ],
)

_SEL = None


def _sel_const():
    global _SEL
    if _SEL is None:
        import numpy as _np
        m = _np.zeros((8, _D, 128), _np.float32)
        for s in range(8):
            for d in range(_D):
                m[s, d, 16 * s + d] = 1.0
        _SEL = jnp.asarray(m)
    return _SEL


def kernel(user, event, user_emb, event_emb, W1, b1, W2, b2):
    utab, etab = _tc_repack(user_emb.T, event_emb.T, _sel_const())
    params = jnp.concatenate(
        [W1.reshape(-1), b1.reshape(-1), W2.reshape(-1), b2.reshape(-1)])
    splats = jnp.broadcast_to(params.reshape(-1, 1), (params.shape[0], 16))
    wpack = jnp.pad(splats.reshape(-1), (0, 72 * 128 - params.shape[0] * 16))
    wpack = wpack.reshape(72, 128)
    out = _sc_ncf(user, event, utab, etab, wpack)
    return out.reshape(_B, 1)


# speedup vs baseline: 4.3195x; 1.7732x over previous
import functools

import jax
import jax.numpy as jnp
from jax import lax
from jax.experimental import pallas as pl
from jax.experimental.pallas import tpu as pltpu
from jax.experimental.pallas import tpu_sc as plsc

_B = 16384
_D = 16
_H = 16
_NC = 2
_NS = 16
_NW = _NC * _NS
_BPW = _B // _NW
_CHUNK = 128
_NCH = _BPW // _CHUNK
_GPC = _CHUNK // 16
_NSLAB = 123 * 1024


def _wrow(m):
    return m // 8, (m % 8) * 16


def _sc_body(uidx_hbm, eidx_hbm, utab_hbm, etab_hbm, wpack_hbm, out_hbm,
             uidx_v, eidx_v, uslab_v, eslab_v,
             ubufA, ubufB, ebufA, ebufB, wpack_v, out_v,
             semA_u, semB_u, semA_e, semB_e):
    wid = lax.axis_index("s") * _NC + lax.axis_index("c")
    base = wid * _BPW
    pltpu.sync_copy(uidx_hbm.at[pl.ds(base, _BPW)], uidx_v)
    pltpu.sync_copy(eidx_hbm.at[pl.ds(base, _BPW)], eidx_v)
    pltpu.sync_copy(wpack_hbm, wpack_v)
    for t in range(_BPW // 16):
        sl = pl.ds(t * 16, 16)
        u = uidx_v[sl]
        e = eidx_v[sl]
        uslab_v[sl] = lax.shift_left(
            lax.shift_right_logical(u, 13), 10) + (u & 1023)
        eslab_v[sl] = lax.shift_left(
            lax.shift_right_logical(e, 13), 10) + (e & 1023)

    ubufs = (ubufA, ubufB)
    ebufs = (ebufA, ebufB)
    usems = (semA_u, semB_u)
    esems = (semA_e, semB_e)

    def fire(c):
        sl = pl.ds(c * _CHUNK, _CHUNK)
        cu = pltpu.async_copy(
            utab_hbm.at[uslab_v.at[sl]], ubufs[c % 2], usems[c % 2])
        ce = pltpu.async_copy(
            etab_hbm.at[eslab_v.at[sl]], ebufs[c % 2], esems[c % 2])
        return cu, ce

    inflight = fire(0)

    def make_group(c):
        ubuf, ebuf = ubufs[c % 2], ebufs[c % 2]

        def group(g, _):
            s0 = c * _CHUNK + g * 16
            lrow = g * 16 + lax.iota(jnp.int32, 16)
            idxu = uidx_v[pl.ds(s0, 16)]
            idxe = eidx_v[pl.ds(s0, 16)]
            cu = (lax.shift_right_logical(idxu, 10) & 7) * 16
            ce = (lax.shift_right_logical(idxe, 10) & 7) * 16
            xs = []
            for k in range(_D):
                xs.append(plsc.load_gather(ubuf, [lrow, cu + k]))
            for k in range(_D):
                xs.append(plsc.load_gather(ebuf, [lrow, ce + k]))
            r, c0 = _wrow(544)
            o = wpack_v[r, pl.ds(c0, 16)]
            for j in range(_H):
                r, c0 = _wrow(512 + j)
                acc = wpack_v[r, pl.ds(c0, 16)]
                for k in range(2 * _D):
                    r, c0 = _wrow(j * 2 * _D + k)
                    acc = acc + wpack_v[r, pl.ds(c0, 16)] * xs[k]
                r, c0 = _wrow(528 + j)
                o = o + wpack_v[r, pl.ds(c0, 16)] * jnp.maximum(acc, 0.0)
            out_v[pl.ds(s0, 16)] = 1.0 / (1.0 + jnp.exp(-o))
            return 0

        return group

    for c in range(_NCH):
        cu, ce = inflight
        if c + 1 < _NCH:
            nxt = fire(c + 1)
        cu.wait()
        ce.wait()
        if c + 1 < _NCH:
            inflight = nxt
        lax.fori_loop(0, _GPC, make_group(c), 0)

    pltpu.sync_copy(out_v, out_hbm.at[pl.ds(base, _BPW)])


_sc_ncf = functools.partial(
    pl.kernel,
    out_type=jax.ShapeDtypeStruct((_B,), jnp.float32),
    mesh=plsc.VectorSubcoreMesh(core_axis_name="c", subcore_axis_name="s"),
    compiler_params=pltpu.CompilerParams(needs_layout_passes=False),
    scratch_types=[
        pltpu.VMEM((_BPW,), jnp.int32),
        pltpu.VMEM((_BPW,), jnp.int32),
        pltpu.VMEM((_BPW,), jnp.int32),
        pltpu.VMEM((_BPW,), jnp.int32),
        pltpu.VMEM((_CHUNK, 128), jnp.float32),
        pltpu.VMEM((_CHUNK, 128), jnp.float32),
        pltpu.VMEM((_CHUNK, 128), jnp.float32),
        pltpu.VMEM((_CHUNK, 128), jnp.float32),
        pltpu.VMEM((72, 128), jnp.float32),
        pltpu.VMEM((_BPW,), jnp.float32),
        pltpu.SemaphoreType.DMA,
        pltpu.SemaphoreType.DMA,
        pltpu.SemaphoreType.DMA,
        pltpu.SemaphoreType.DMA,
    ],
)(_sc_body)


_RC = 8192
_RB = (1000000 + _RC - 1) // _RC


def _repack_body(ut_ref, et_ref, sel_ref, uo_ref, eo_ref):
    def one(t_ref, o_ref):
        x = t_ref[...]
        q = _RC // 8
        xcat = jnp.concatenate(
            [x[:, s * q:(s + 1) * q] for s in range(8)], axis=0)
        o_ref[...] = xcat.T

    one(ut_ref, uo_ref)
    one(et_ref, eo_ref)


_tc_repack = pl.pallas_call(
    _repack_body,
    grid=(_RB,),
    compiler_params=pltpu.CompilerParams(fuse_transposed_lhs_in_matmul=True),
    in_specs=[
        pl.BlockSpec((_D, _RC), lambda i: (0, i)),
        pl.BlockSpec((_D, _RC), lambda i: (0, i)),
        pl.BlockSpec((8, _D, 128), lambda i: (0, 0, 0)),
    ],
    out_specs=[
        pl.BlockSpec((_RC // 8, 128), lambda i: (i, 0)),
        pl.BlockSpec((_RC // 8, 128), lambda i: (i, 0)),
    ],
    out_shape=[
        jax.ShapeDtypeStruct((_NSLAB, 128), jnp.float32),
        jax.ShapeDtypeStruct((_NSLAB, 128), jnp.float32),
    ],
)

_SEL = None


def _sel_const():
    global _SEL
    if _SEL is None:
        import numpy as _np
        m = _np.zeros((8, _D, 128), _np.float32)
        for s in range(8):
            for d in range(_D):
                m[s, d, 16 * s + d] = 1.0
        _SEL = jnp.asarray(m)
    return _SEL


def kernel(user, event, user_emb, event_emb, W1, b1, W2, b2):
    utab, etab = _tc_repack(user_emb.T, event_emb.T, _sel_const())
    params = jnp.concatenate(
        [W1.reshape(-1), b1.reshape(-1), W2.reshape(-1), b2.reshape(-1)])
    splats = jnp.broadcast_to(params.reshape(-1, 1), (params.shape[0], 16))
    wpack = jnp.pad(splats.reshape(-1), (0, 72 * 128 - params.shape[0] * 16))
    wpack = wpack.reshape(72, 128)
    out = _sc_ncf(user, event, utab, etab, wpack)
    return out.reshape(_B, 1)


# speedup vs baseline: 4.6279x; 1.0714x over previous
import functools

import jax
import jax.numpy as jnp
from jax import lax
from jax.experimental import pallas as pl
from jax.experimental.pallas import tpu as pltpu
from jax.experimental.pallas import tpu_sc as plsc

_B = 16384
_D = 16
_H = 16
_NC = 2
_NS = 16
_NW = _NC * _NS
_BPW = _B // _NW
_CHUNK = 128
_NCH = _BPW // _CHUNK
_GPC = _CHUNK // 16
_NSLAB = 123 * 1024


def _wrow(m):
    return m // 8, (m % 8) * 16


def _sc_body(uidx_hbm, eidx_hbm, utab_hbm, etab_hbm, wpack_hbm, out_hbm,
             uidx_v, eidx_v, uslab_v, eslab_v,
             ubufA, ubufB, ebufA, ebufB, wpack_v, out_v,
             semA_u, semB_u, semA_e, semB_e):
    wid = lax.axis_index("s") * _NC + lax.axis_index("c")
    base = wid * _BPW
    pltpu.sync_copy(uidx_hbm.at[pl.ds(base, _BPW)], uidx_v)
    pltpu.sync_copy(eidx_hbm.at[pl.ds(base, _BPW)], eidx_v)
    pltpu.sync_copy(wpack_hbm, wpack_v)
    for t in range(_BPW // 16):
        sl = pl.ds(t * 16, 16)
        u = uidx_v[sl]
        e = eidx_v[sl]
        uslab_v[sl] = lax.shift_left(
            lax.shift_right_logical(u, 13), 10) + (u & 1023)
        eslab_v[sl] = lax.shift_left(
            lax.shift_right_logical(e, 13), 10) + (e & 1023)

    ubufs = (ubufA, ubufB)
    ebufs = (ebufA, ebufB)
    usems = (semA_u, semB_u)
    esems = (semA_e, semB_e)

    b1v = []
    w2v = []
    for j in range(_H):
        r, c0 = _wrow(512 + j)
        b1v.append(wpack_v[r, pl.ds(c0, 16)])
        r, c0 = _wrow(528 + j)
        w2v.append(wpack_v[r, pl.ds(c0, 16)])
    r, c0 = _wrow(544)
    o0 = wpack_v[r, pl.ds(c0, 16)]

    def fire(c):
        sl = pl.ds(c * _CHUNK, _CHUNK)
        cu = pltpu.async_copy(
            utab_hbm.at[uslab_v.at[sl]], ubufs[c % 2], usems[c % 2])
        ce = pltpu.async_copy(
            etab_hbm.at[eslab_v.at[sl]], ebufs[c % 2], esems[c % 2])
        return cu, ce

    inflight = fire(0)

    def make_group(c):
        ubuf, ebuf = ubufs[c % 2], ebufs[c % 2]

        def group(g, _):
            s0 = c * _CHUNK + g * 16
            lrow = g * 16 + lax.iota(jnp.int32, 16)
            idxu = uidx_v[pl.ds(s0, 16)]
            idxe = eidx_v[pl.ds(s0, 16)]
            cu = (lax.shift_right_logical(idxu, 10) & 7) * 16
            ce = (lax.shift_right_logical(idxe, 10) & 7) * 16
            o = o0
            for j in range(_H):
                hu = plsc.load_gather(ubuf, [lrow, cu + j])
                he = plsc.load_gather(ebuf, [lrow, ce + j])
                h = hu + he + b1v[j]
                o = o + w2v[j] * jnp.maximum(h, 0.0)
            out_v[pl.ds(s0, 16)] = 1.0 / (1.0 + jnp.exp(-o))
            return 0

        return group

    for c in range(_NCH):
        cu, ce = inflight
        if c + 1 < _NCH:
            nxt = fire(c + 1)
        cu.wait()
        ce.wait()
        if c + 1 < _NCH:
            inflight = nxt
        lax.fori_loop(0, _GPC, make_group(c), 0)

    pltpu.sync_copy(out_v, out_hbm.at[pl.ds(base, _BPW)])


_sc_ncf = functools.partial(
    pl.kernel,
    out_type=jax.ShapeDtypeStruct((_B,), jnp.float32),
    mesh=plsc.VectorSubcoreMesh(core_axis_name="c", subcore_axis_name="s"),
    compiler_params=pltpu.CompilerParams(needs_layout_passes=False),
    scratch_types=[
        pltpu.VMEM((_BPW,), jnp.int32),
        pltpu.VMEM((_BPW,), jnp.int32),
        pltpu.VMEM((_BPW,), jnp.int32),
        pltpu.VMEM((_BPW,), jnp.int32),
        pltpu.VMEM((_CHUNK, 128), jnp.float32),
        pltpu.VMEM((_CHUNK, 128), jnp.float32),
        pltpu.VMEM((_CHUNK, 128), jnp.float32),
        pltpu.VMEM((_CHUNK, 128), jnp.float32),
        pltpu.VMEM((72, 128), jnp.float32),
        pltpu.VMEM((_BPW,), jnp.float32),
        pltpu.SemaphoreType.DMA,
        pltpu.SemaphoreType.DMA,
        pltpu.SemaphoreType.DMA,
        pltpu.SemaphoreType.DMA,
    ],
)(_sc_body)


_RC = 8192
_RB = (1000000 + _RC - 1) // _RC


def _repack_body(ut_ref, et_ref, bd_ref, uo_ref, eo_ref):
    def one(t_ref, bd, o_ref):
        x = t_ref[...]
        q = _RC // 8
        xcat = jnp.concatenate(
            [x[:, s * q:(s + 1) * q] for s in range(8)], axis=0)
        o_ref[...] = jnp.dot(bd, xcat,
                             preferred_element_type=jnp.float32).T

    one(ut_ref, bd_ref[0], uo_ref)
    one(et_ref, bd_ref[1], eo_ref)


_tc_repack = pl.pallas_call(
    _repack_body,
    grid=(_RB,),
    compiler_params=pltpu.CompilerParams(fuse_transposed_lhs_in_matmul=True),
    in_specs=[
        pl.BlockSpec((_D, _RC), lambda i: (0, i)),
        pl.BlockSpec((_D, _RC), lambda i: (0, i)),
        pl.BlockSpec((2, 128, 128), lambda i: (0, 0, 0)),
    ],
    out_specs=[
        pl.BlockSpec((_RC // 8, 128), lambda i: (i, 0)),
        pl.BlockSpec((_RC // 8, 128), lambda i: (i, 0)),
    ],
    out_shape=[
        jax.ShapeDtypeStruct((_NSLAB, 128), jnp.float32),
        jax.ShapeDtypeStruct((_NSLAB, 128), jnp.float32),
    ],
)


def kernel(user, event, user_emb, event_emb, W1, b1, W2, b2):
    eye8 = jnp.eye(8, dtype=jnp.float32)
    bd = jnp.stack([jnp.kron(eye8, W1[:, :_D]),
                    jnp.kron(eye8, W1[:, _D:])])
    utab, etab = _tc_repack(user_emb.T, event_emb.T, bd)
    params = jnp.concatenate(
        [W1.reshape(-1), b1.reshape(-1), W2.reshape(-1), b2.reshape(-1)])
    splats = jnp.broadcast_to(params.reshape(-1, 1), (params.shape[0], 16))
    wpack = jnp.pad(splats.reshape(-1), (0, 72 * 128 - params.shape[0] * 16))
    wpack = wpack.reshape(72, 128)
    out = _sc_ncf(user, event, utab, etab, wpack)
    return out.reshape(_B, 1)


# speedup vs baseline: 7.0832x; 1.5305x over previous
import functools

import jax
import jax.numpy as jnp
from jax import lax
from jax.experimental import pallas as pl
from jax.experimental.pallas import tpu as pltpu
from jax.experimental.pallas import tpu_sc as plsc

_B = 16384
_D = 16
_H = 16
_NC = 2
_NS = 16
_NW = _NC * _NS
_BPW = _B // _NW
_CHUNK = 128
_NCH = _BPW // _CHUNK
_GPC = _CHUNK // 16
_RC = 32768
_RB = (1000000 + _RC - 1) // _RC
_Q = _RC // 8
_RSH = _RC.bit_length() - 1
_QSH = _Q.bit_length() - 1
_NSLAB = _RB * _Q


def _wrow(m):
    return m // 8, (m % 8) * 16


def _sc_body(uidx_hbm, eidx_hbm, utab_hbm, etab_hbm, wpack_hbm, out_hbm,
             uidx_v, eidx_v, uslab_v, eslab_v,
             ubufA, ubufB, ebufA, ebufB, wpack_v, out_v,
             semA_u, semB_u, semA_e, semB_e):
    wid = lax.axis_index("s") * _NC + lax.axis_index("c")
    base = wid * _BPW
    pltpu.sync_copy(uidx_hbm.at[pl.ds(base, _BPW)], uidx_v)
    pltpu.sync_copy(eidx_hbm.at[pl.ds(base, _BPW)], eidx_v)
    pltpu.sync_copy(wpack_hbm, wpack_v)
    for t in range(_BPW // 16):
        sl = pl.ds(t * 16, 16)
        u = uidx_v[sl]
        e = eidx_v[sl]
        uslab_v[sl] = lax.shift_left(
            lax.shift_right_logical(u, _RSH), _QSH) + (u & (_Q - 1))
        eslab_v[sl] = lax.shift_left(
            lax.shift_right_logical(e, _RSH), _QSH) + (e & (_Q - 1))

    ubufs = (ubufA, ubufB)
    ebufs = (ebufA, ebufB)
    usems = (semA_u, semB_u)
    esems = (semA_e, semB_e)

    b1v = []
    w2v = []
    for j in range(_H):
        r, c0 = _wrow(512 + j)
        b1v.append(wpack_v[r, pl.ds(c0, 16)])
        r, c0 = _wrow(528 + j)
        w2v.append(wpack_v[r, pl.ds(c0, 16)])
    r, c0 = _wrow(544)
    o0 = wpack_v[r, pl.ds(c0, 16)]

    def fire(c):
        sl = pl.ds(c * _CHUNK, _CHUNK)
        cu = pltpu.async_copy(
            utab_hbm.at[uslab_v.at[sl]], ubufs[c % 2], usems[c % 2])
        ce = pltpu.async_copy(
            etab_hbm.at[eslab_v.at[sl]], ebufs[c % 2], esems[c % 2])
        return cu, ce

    inflight = fire(0)

    def make_group(c):
        ubuf, ebuf = ubufs[c % 2], ebufs[c % 2]

        def group(g, _):
            s0 = c * _CHUNK + g * 16
            lrow = g * 16 + lax.iota(jnp.int32, 16)
            idxu = uidx_v[pl.ds(s0, 16)]
            idxe = eidx_v[pl.ds(s0, 16)]
            cu = (lax.shift_right_logical(idxu, _QSH) & 7) * 16
            ce = (lax.shift_right_logical(idxe, _QSH) & 7) * 16
            o = o0
            for j in range(_H):
                hu = plsc.load_gather(ubuf, [lrow, cu + j])
                he = plsc.load_gather(ebuf, [lrow, ce + j])
                h = hu + he + b1v[j]
                o = o + w2v[j] * jnp.maximum(h, 0.0)
            out_v[pl.ds(s0, 16)] = 1.0 / (1.0 + jnp.exp(-o))
            return 0

        return group

    for c in range(_NCH):
        cu, ce = inflight
        if c + 1 < _NCH:
            nxt = fire(c + 1)
        cu.wait()
        ce.wait()
        if c + 1 < _NCH:
            inflight = nxt
        lax.fori_loop(0, _GPC, make_group(c), 0)

    pltpu.sync_copy(out_v, out_hbm.at[pl.ds(base, _BPW)])


_sc_ncf = functools.partial(
    pl.kernel,
    out_type=jax.ShapeDtypeStruct((_B,), jnp.float32),
    mesh=plsc.VectorSubcoreMesh(core_axis_name="c", subcore_axis_name="s"),
    compiler_params=pltpu.CompilerParams(needs_layout_passes=False),
    scratch_types=[
        pltpu.VMEM((_BPW,), jnp.int32),
        pltpu.VMEM((_BPW,), jnp.int32),
        pltpu.VMEM((_BPW,), jnp.int32),
        pltpu.VMEM((_BPW,), jnp.int32),
        pltpu.VMEM((_CHUNK, 128), jnp.float32),
        pltpu.VMEM((_CHUNK, 128), jnp.float32),
        pltpu.VMEM((_CHUNK, 128), jnp.float32),
        pltpu.VMEM((_CHUNK, 128), jnp.float32),
        pltpu.VMEM((72, 128), jnp.float32),
        pltpu.VMEM((_BPW,), jnp.float32),
        pltpu.SemaphoreType.DMA,
        pltpu.SemaphoreType.DMA,
        pltpu.SemaphoreType.DMA,
        pltpu.SemaphoreType.DMA,
    ],
)(_sc_body)


def _repack_body(ut_ref, et_ref, bd_ref, uo_ref, eo_ref):
    def one(t_ref, bd, o_ref):
        x = t_ref[...]
        q = _RC // 8
        xcat = jnp.concatenate(
            [x[:, s * q:(s + 1) * q] for s in range(8)], axis=0)
        o_ref[...] = jnp.dot(bd, xcat,
                             preferred_element_type=jnp.float32).T

    one(ut_ref, bd_ref[0], uo_ref)
    one(et_ref, bd_ref[1], eo_ref)


_tc_repack = pl.pallas_call(
    _repack_body,
    grid=(_RB,),
    compiler_params=pltpu.CompilerParams(fuse_transposed_lhs_in_matmul=True),
    in_specs=[
        pl.BlockSpec((_D, _RC), lambda i: (0, i)),
        pl.BlockSpec((_D, _RC), lambda i: (0, i)),
        pl.BlockSpec((2, 128, 128), lambda i: (0, 0, 0)),
    ],
    out_specs=[
        pl.BlockSpec((_RC // 8, 128), lambda i: (i, 0)),
        pl.BlockSpec((_RC // 8, 128), lambda i: (i, 0)),
    ],
    out_shape=[
        jax.ShapeDtypeStruct((_NSLAB, 128), jnp.float32),
        jax.ShapeDtypeStruct((_NSLAB, 128), jnp.float32),
    ],
)


def kernel(user, event, user_emb, event_emb, W1, b1, W2, b2):
    eye8 = jnp.eye(8, dtype=jnp.float32)
    bd = jnp.stack([jnp.kron(eye8, W1[:, :_D]),
                    jnp.kron(eye8, W1[:, _D:])])
    utab, etab = _tc_repack(user_emb.T, event_emb.T, bd)
    params = jnp.concatenate(
        [W1.reshape(-1), b1.reshape(-1), W2.reshape(-1), b2.reshape(-1)])
    splats = jnp.broadcast_to(params.reshape(-1, 1), (params.shape[0], 16))
    wpack = jnp.pad(splats.reshape(-1), (0, 72 * 128 - params.shape[0] * 16))
    wpack = wpack.reshape(72, 128)
    out = _sc_ncf(user, event, utab, etab, wpack)
    return out.reshape(_B, 1)


# speedup vs baseline: 7.2765x; 1.0273x over previous
import functools

import jax
import jax.numpy as jnp
from jax import lax
from jax.experimental import pallas as pl
from jax.experimental.pallas import tpu as pltpu
from jax.experimental.pallas import tpu_sc as plsc

_B = 16384
_D = 16
_H = 16
_NC = 2
_NS = 16
_NW = _NC * _NS
_BPW = _B // _NW
_CHUNK = 128
_NCH = _BPW // _CHUNK
_GPC = _CHUNK // 16
_RC = 65536
_RB = (1000000 + _RC - 1) // _RC
_Q = _RC // 8
_RSH = _RC.bit_length() - 1
_QSH = _Q.bit_length() - 1
_NSLAB = _RB * _Q


def _wrow(m):
    return m // 8, (m % 8) * 16


def _sc_body(uidx_hbm, eidx_hbm, utab_hbm, etab_hbm, wpack_hbm, out_hbm,
             uidx_v, eidx_v, uslab_v, eslab_v,
             ubufA, ubufB, ebufA, ebufB, wpack_v, out_v,
             semA_u, semB_u, semA_e, semB_e):
    wid = lax.axis_index("s") * _NC + lax.axis_index("c")
    base = wid * _BPW
    pltpu.sync_copy(uidx_hbm.at[pl.ds(base, _BPW)], uidx_v)
    pltpu.sync_copy(eidx_hbm.at[pl.ds(base, _BPW)], eidx_v)
    pltpu.sync_copy(wpack_hbm, wpack_v)
    for t in range(_BPW // 16):
        sl = pl.ds(t * 16, 16)
        u = uidx_v[sl]
        e = eidx_v[sl]
        uslab_v[sl] = lax.shift_left(
            lax.shift_right_logical(u, _RSH), _QSH) + (u & (_Q - 1))
        eslab_v[sl] = lax.shift_left(
            lax.shift_right_logical(e, _RSH), _QSH) + (e & (_Q - 1))

    ubufs = (ubufA, ubufB)
    ebufs = (ebufA, ebufB)
    usems = (semA_u, semB_u)
    esems = (semA_e, semB_e)

    b1v = []
    w2v = []
    for j in range(_H):
        r, c0 = _wrow(512 + j)
        b1v.append(wpack_v[r, pl.ds(c0, 16)])
        r, c0 = _wrow(528 + j)
        w2v.append(wpack_v[r, pl.ds(c0, 16)])
    r, c0 = _wrow(544)
    o0 = wpack_v[r, pl.ds(c0, 16)]

    def fire(c):
        sl = pl.ds(c * _CHUNK, _CHUNK)
        cu = pltpu.async_copy(
            utab_hbm.at[uslab_v.at[sl]], ubufs[c % 2], usems[c % 2])
        ce = pltpu.async_copy(
            etab_hbm.at[eslab_v.at[sl]], ebufs[c % 2], esems[c % 2])
        return cu, ce

    inflight = fire(0)

    def make_group(c):
        ubuf, ebuf = ubufs[c % 2], ebufs[c % 2]

        def group(g, _):
            s0 = c * _CHUNK + g * 16
            lrow = g * 16 + lax.iota(jnp.int32, 16)
            idxu = uidx_v[pl.ds(s0, 16)]
            idxe = eidx_v[pl.ds(s0, 16)]
            cu = (lax.shift_right_logical(idxu, _QSH) & 7) * 16
            ce = (lax.shift_right_logical(idxe, _QSH) & 7) * 16
            o = o0
            for j in range(_H):
                hu = plsc.load_gather(ubuf, [lrow, cu + j])
                he = plsc.load_gather(ebuf, [lrow, ce + j])
                h = hu + he + b1v[j]
                o = o + w2v[j] * jnp.maximum(h, 0.0)
            out_v[pl.ds(s0, 16)] = 1.0 / (1.0 + jnp.exp(-o))
            return 0

        return group

    for c in range(_NCH):
        cu, ce = inflight
        if c + 1 < _NCH:
            nxt = fire(c + 1)
        cu.wait()
        ce.wait()
        if c + 1 < _NCH:
            inflight = nxt
        lax.fori_loop(0, _GPC, make_group(c), 0)

    pltpu.sync_copy(out_v, out_hbm.at[pl.ds(base, _BPW)])


_sc_ncf = functools.partial(
    pl.kernel,
    out_type=jax.ShapeDtypeStruct((_B,), jnp.float32),
    mesh=plsc.VectorSubcoreMesh(core_axis_name="c", subcore_axis_name="s"),
    compiler_params=pltpu.CompilerParams(needs_layout_passes=False),
    scratch_types=[
        pltpu.VMEM((_BPW,), jnp.int32),
        pltpu.VMEM((_BPW,), jnp.int32),
        pltpu.VMEM((_BPW,), jnp.int32),
        pltpu.VMEM((_BPW,), jnp.int32),
        pltpu.VMEM((_CHUNK, 128), jnp.float32),
        pltpu.VMEM((_CHUNK, 128), jnp.float32),
        pltpu.VMEM((_CHUNK, 128), jnp.float32),
        pltpu.VMEM((_CHUNK, 128), jnp.float32),
        pltpu.VMEM((72, 128), jnp.float32),
        pltpu.VMEM((_BPW,), jnp.float32),
        pltpu.SemaphoreType.DMA,
        pltpu.SemaphoreType.DMA,
        pltpu.SemaphoreType.DMA,
        pltpu.SemaphoreType.DMA,
    ],
)(_sc_body)


def _repack_body(ut_ref, et_ref, bd_ref, uo_ref, eo_ref):
    def one(t_ref, bd, o_ref):
        x = t_ref[...]
        q = _RC // 8
        xcat = jnp.concatenate(
            [x[:, s * q:(s + 1) * q] for s in range(8)], axis=0)
        o_ref[...] = jnp.dot(bd, xcat,
                             preferred_element_type=jnp.float32).T

    one(ut_ref, bd_ref[0], uo_ref)
    one(et_ref, bd_ref[1], eo_ref)


_tc_repack = pl.pallas_call(
    _repack_body,
    grid=(_RB,),
    compiler_params=pltpu.CompilerParams(fuse_transposed_lhs_in_matmul=True),
    in_specs=[
        pl.BlockSpec((_D, _RC), lambda i: (0, i)),
        pl.BlockSpec((_D, _RC), lambda i: (0, i)),
        pl.BlockSpec((2, 128, 128), lambda i: (0, 0, 0)),
    ],
    out_specs=[
        pl.BlockSpec((_RC // 8, 128), lambda i: (i, 0)),
        pl.BlockSpec((_RC // 8, 128), lambda i: (i, 0)),
    ],
    out_shape=[
        jax.ShapeDtypeStruct((_NSLAB, 128), jnp.float32),
        jax.ShapeDtypeStruct((_NSLAB, 128), jnp.float32),
    ],
)


def kernel(user, event, user_emb, event_emb, W1, b1, W2, b2):
    eye8 = jnp.eye(8, dtype=jnp.float32)
    bd = jnp.stack([jnp.kron(eye8, W1[:, :_D]),
                    jnp.kron(eye8, W1[:, _D:])])
    utab, etab = _tc_repack(user_emb.T, event_emb.T, bd)
    params = jnp.concatenate(
        [W1.reshape(-1), b1.reshape(-1), W2.reshape(-1), b2.reshape(-1)])
    splats = jnp.broadcast_to(params.reshape(-1, 1), (params.shape[0], 16))
    wpack = jnp.pad(splats.reshape(-1), (0, 72 * 128 - params.shape[0] * 16))
    wpack = wpack.reshape(72, 128)
    out = _sc_ncf(user, event, utab, etab, wpack)
    return out.reshape(_B, 1)
